# baseline (speedup 1.0000x reference)
import functools

import jax
import jax.numpy as jnp
from jax import lax
from jax.experimental import pallas as pl
from jax.experimental.pallas import tpu as pltpu
from jax.experimental.pallas import tpu_sc as plsc

_NC = 2
_NS = 16
_NW = _NC * _NS
_L = 16

_EMBED = 64
_BLK = 64
_C = 32768
_Q0 = 4096


def _tc_scores_body(x_ref, w_ref, o_ref):
    x = x_ref[...]
    w = w_ref[...]
    s = jnp.sum(x * w, axis=0)
    o_ref[...] = s.reshape(_C // 128, 128)


def _tc_scores(tab_t, w, n_rows):
    grid = n_rows * 128 // _C
    return pl.pallas_call(
        _tc_scores_body,
        grid=(grid,),
        in_specs=[
            pl.BlockSpec((_EMBED, _C), lambda i: (0, i)),
            pl.BlockSpec((_EMBED, 1), lambda i: (0, 0)),
        ],
        out_specs=pl.BlockSpec((_C // 128, 128), lambda i: (i, 0)),
        out_shape=jax.ShapeDtypeStruct((n_rows, 128), jnp.float32),
    )(tab_t, w)


def _sc_matvec_body(utab_t, mtab_t, wrep2, su_hi, sm_hi,
                    bu, bm, wv, rowbuf,
                    sem_u0, sem_u1, sem_m0, sem_m1,
                    *, nq, nq_hi):
    wid = lax.axis_index("s") * _NC + lax.axis_index("c")
    pltpu.sync_copy(wrep2, wv)

    sems_u = (sem_u0, sem_u1)
    sems_m = (sem_m0, sem_m1)
    n_per = -(-nq_hi // _NW)
    n_k = n_per + (n_per % 2)
    max_q = nq - 1

    def qid_of(k):
        return jnp.minimum(_Q0 + wid + 32 * k, max_q)

    def fire(k, s):
        col0 = pl.multiple_of(qid_of(k) * 128, 128)
        sl = pl.ds(col0, 128)
        pltpu.async_copy(utab_t.at[:, sl], bu.at[s], sems_u[s])
        pltpu.async_copy(mtab_t.at[:, sl], bm.at[s], sems_m[s])

    def drain(s):
        pltpu.make_async_copy(utab_t.at[:, pl.ds(0, 128)], bu.at[s],
                              sems_u[s]).wait()
        pltpu.make_async_copy(mtab_t.at[:, pl.ds(0, 128)], bm.at[s],
                              sems_m[s]).wait()

    def compute(k, s):
        row = qid_of(k) - _Q0
        for tbl, buf, w_off, dst in ((0, bu, 0, su_hi), (1, bm, 1024, sm_hi)):
            accs = [jnp.zeros((_L,), jnp.float32) for _ in range(8)]
            for d in range(_EMBED):
                w_d = wv[pl.ds(w_off + d * _L, _L)]
                for g in range(8):
                    accs[g] = accs[g] + buf[s, d, pl.ds(g * _L, _L)] * w_d
            for g in range(8):
                rowbuf[pl.ds(g * _L, _L)] = accs[g]
            pltpu.sync_copy(rowbuf, dst.at[row])

    fire(0, 0)

    def superstep(i, carry):
        k0 = 2 * i
        fire(k0 + 1, 1)
        drain(0)
        compute(k0, 0)

        @pl.when(k0 + 2 < n_k)
        def _():
            fire(k0 + 2, 0)

        drain(1)
        compute(k0 + 1, 1)
        return carry

    lax.fori_loop(0, n_k // 2, superstep, 0)


def _sc_matvec(utab_t, mtab_t, wrep2, nq):
    nq_hi = nq - _Q0
    mesh = plsc.VectorSubcoreMesh(core_axis_name="c", subcore_axis_name="s")
    body = functools.partial(_sc_matvec_body, nq=nq, nq_hi=nq_hi)
    run = pl.kernel(
        body,
        out_type=(
            jax.ShapeDtypeStruct((nq_hi, 128), jnp.float32),
            jax.ShapeDtypeStruct((nq_hi, 128), jnp.float32),
        ),
        mesh=mesh,
        compiler_params=pltpu.CompilerParams(needs_layout_passes=False),
        scratch_types=[
            pltpu.VMEM((2, _EMBED, 128), jnp.float32),
            pltpu.VMEM((2, _EMBED, 128), jnp.float32),
            pltpu.VMEM((2 * _EMBED * _L,), jnp.float32),
            pltpu.VMEM((128,), jnp.float32),
            pltpu.SemaphoreType.DMA,
            pltpu.SemaphoreType.DMA,
            pltpu.SemaphoreType.DMA,
            pltpu.SemaphoreType.DMA,
        ],
    )
    return run(utab_t, mtab_t, wrep2)


def _sc_body(users_r, movies_r, su, sm, brep, out,
             qu_v, tu_v, qm_v, tm_v, rows_u, rows_m, out_v, b_v,
             sem_u0, sem_u1, sem_m0, sem_m1,
             *, b_per_w, n_blocks):
    wid = lax.axis_index("s") * _NC + lax.axis_index("c")
    chunks_per_blk = _BLK // _L

    pltpu.sync_copy(users_r.at[wid], qu_v)
    pltpu.sync_copy(movies_r.at[wid], qm_v)
    pltpu.sync_copy(brep, b_v)

    def split(i, carry):
        sl = pl.ds(i * _L, _L)
        u = qu_v[sl]
        m = qm_v[sl]
        tu_v[sl] = u & 127
        qu_v[sl] = u >> 7
        tm_v[sl] = m & 127
        qm_v[sl] = m >> 7
        return carry

    lax.fori_loop(0, b_per_w // _L, split, 0)

    sems_u = (sem_u0, sem_u1)
    sems_m = (sem_m0, sem_m1)

    def fire(j):
        s = j % 2
        blk = pl.ds(j * _BLK, _BLK)
        pltpu.async_copy(su.at[qu_v.at[blk]], rows_u.at[s], sems_u[s])
        pltpu.async_copy(sm.at[qm_v.at[blk]], rows_m.at[s], sems_m[s])

    def drain(j):
        s = j % 2
        pltpu.make_async_copy(su.at[pl.ds(0, _BLK)], rows_u.at[s],
                              sems_u[s]).wait()
        pltpu.make_async_copy(sm.at[pl.ds(0, _BLK)], rows_m.at[s],
                              sems_m[s]).wait()

    row16 = lax.iota(jnp.int32, _L)
    bias = b_v[...]

    fire(0)
    fire(1)
    for j in range(n_blocks):
        s = j % 2
        drain(j)
        ru = rows_u.at[s]
        rm = rows_m.at[s]

        def chunk(k, carry, *, j=j, ru=ru, rm=rm):
            g = j * chunks_per_blk + k
            rid = k * _L + row16
            t_u = tu_v[pl.ds(g * _L, _L)]
            t_m = tm_v[pl.ds(g * _L, _L)]
            vu = plsc.load_gather(ru, [rid, t_u])
            vm = plsc.load_gather(rm, [rid, t_m])
            out_v[pl.ds(g * _L, _L)] = vu + vm + bias
            return carry

        lax.fori_loop(0, chunks_per_blk, chunk, 0)
        if j + 2 < n_blocks:
            fire(j + 2)

    pltpu.sync_copy(out_v, out.at[wid])


def kernel(users, movies, user_table, movie_table, W, b):
    B = users.shape[0]
    assert B % (_NW * _BLK) == 0
    b_per_w = B // _NW
    n_blocks = b_per_w // _BLK

    users_r = users.astype(jnp.int32).reshape(_NW, b_per_w)
    movies_r = movies.astype(jnp.int32).reshape(_NW, b_per_w)

    utab_t = user_table.T
    mtab_t = movie_table.T
    V = user_table.shape[0]
    nq = -(-V // 128)

    wu = W.reshape(-1)[:_EMBED].astype(jnp.float32).reshape(_EMBED, 1)
    wm = W.reshape(-1)[_EMBED:].astype(jnp.float32).reshape(_EMBED, 1)
    ones = jnp.ones((1, _L), jnp.float32)
    wrep2 = jnp.concatenate(
        [(wu * ones).reshape(-1), (wm * ones).reshape(-1)]
    )

    su_hi, sm_hi = _sc_matvec(utab_t, mtab_t, wrep2, nq)
    su_lo = _tc_scores(utab_t, wu, _Q0)
    sm_lo = _tc_scores(mtab_t, wm, _Q0)

    scores_u = jnp.concatenate([su_lo, su_hi], axis=0)
    scores_m = jnp.concatenate([sm_lo, sm_hi], axis=0)

    brep = jnp.full((_L,), b.reshape(()), dtype=jnp.float32)

    mesh = plsc.VectorSubcoreMesh(core_axis_name="c", subcore_axis_name="s")
    body = functools.partial(_sc_body, b_per_w=b_per_w, n_blocks=n_blocks)
    run = pl.kernel(
        body,
        out_type=jax.ShapeDtypeStruct((_NW, b_per_w), jnp.float32),
        mesh=mesh,
        compiler_params=pltpu.CompilerParams(needs_layout_passes=False),
        scratch_types=[
            pltpu.VMEM((b_per_w,), jnp.int32),
            pltpu.VMEM((b_per_w,), jnp.int32),
            pltpu.VMEM((b_per_w,), jnp.int32),
            pltpu.VMEM((b_per_w,), jnp.int32),
            pltpu.VMEM((2, _BLK, 128), jnp.float32),
            pltpu.VMEM((2, _BLK, 128), jnp.float32),
            pltpu.VMEM((b_per_w,), jnp.float32),
            pltpu.VMEM((_L,), jnp.float32),
            pltpu.SemaphoreType.DMA,
            pltpu.SemaphoreType.DMA,
            pltpu.SemaphoreType.DMA,
            pltpu.SemaphoreType.DMA,
        ],
    )
    out = run(users_r, movies_r, scores_u, scores_m, brep)
    return out.reshape(B, 1)


# speedup vs baseline: 2.0701x; 2.0701x over previous
import functools

import jax
import jax.numpy as jnp
from jax import lax
from jax.experimental import pallas as pl
from jax.experimental.pallas import tpu as pltpu
from jax.experimental.pallas import tpu_sc as plsc

_NC = 2
_NS = 16
_NW = _NC * _NS
_L = 16

_EMBED = 64
_BLK = 64
_C = 32768


def _tc_scores_body(x_ref, w_ref, o_ref):
    x = x_ref[...]
    w = w_ref[...]
    s = jnp.sum(x * w, axis=0)
    o_ref[...] = s.reshape(_C // 128, 128)


def _tc_scores(tab_t, w):
    n = tab_t.shape[1]
    grid = (n + _C - 1) // _C
    return pl.pallas_call(
        _tc_scores_body,
        grid=(grid,),
        in_specs=[
            pl.BlockSpec((_EMBED, _C), lambda i: (0, i)),
            pl.BlockSpec((_EMBED, 1), lambda i: (0, 0)),
        ],
        out_specs=pl.BlockSpec((_C // 128, 128), lambda i: (i, 0)),
        out_shape=jax.ShapeDtypeStruct((grid * (_C // 128), 128), jnp.float32),
    )(tab_t, w)


def _sc_body(users_r, movies_r, su, sm, brep, out,
             qu_v, tu_v, qm_v, tm_v, rows_u, rows_m, out_v, b_v,
             sem_u0, sem_u1, sem_m0, sem_m1,
             *, b_per_w, n_blocks):
    wid = lax.axis_index("s") * _NC + lax.axis_index("c")
    chunks_per_blk = _BLK // _L

    pltpu.sync_copy(users_r.at[wid], qu_v)
    pltpu.sync_copy(movies_r.at[wid], qm_v)
    pltpu.sync_copy(brep, b_v)

    def split(i, carry):
        sl = pl.ds(i * _L, _L)
        u = qu_v[sl]
        m = qm_v[sl]
        tu_v[sl] = u & 127
        qu_v[sl] = u >> 7
        tm_v[sl] = m & 127
        qm_v[sl] = m >> 7
        return carry

    lax.fori_loop(0, b_per_w // _L, split, 0)

    sems_u = (sem_u0, sem_u1)
    sems_m = (sem_m0, sem_m1)

    def fire(j):
        s = j % 2
        blk = pl.ds(j * _BLK, _BLK)
        pltpu.async_copy(su.at[qu_v.at[blk]], rows_u.at[s], sems_u[s])
        pltpu.async_copy(sm.at[qm_v.at[blk]], rows_m.at[s], sems_m[s])

    def drain(j):
        s = j % 2
        pltpu.make_async_copy(su.at[pl.ds(0, _BLK)], rows_u.at[s],
                              sems_u[s]).wait()
        pltpu.make_async_copy(sm.at[pl.ds(0, _BLK)], rows_m.at[s],
                              sems_m[s]).wait()

    row16 = lax.iota(jnp.int32, _L)
    bias = b_v[...]

    fire(0)
    fire(1)
    for j in range(n_blocks):
        s = j % 2
        drain(j)
        ru = rows_u.at[s]
        rm = rows_m.at[s]

        def chunk(k, carry, *, j=j, ru=ru, rm=rm):
            g = j * chunks_per_blk + k
            rid = k * _L + row16
            t_u = tu_v[pl.ds(g * _L, _L)]
            t_m = tm_v[pl.ds(g * _L, _L)]
            vu = plsc.load_gather(ru, [rid, t_u])
            vm = plsc.load_gather(rm, [rid, t_m])
            out_v[pl.ds(g * _L, _L)] = vu + vm + bias
            return carry

        lax.fori_loop(0, chunks_per_blk, chunk, 0)
        if j + 2 < n_blocks:
            fire(j + 2)

    pltpu.sync_copy(out_v, out.at[wid])


def kernel(users, movies, user_table, movie_table, W, b):
    B = users.shape[0]
    assert B % (_NW * _BLK) == 0
    b_per_w = B // _NW
    n_blocks = b_per_w // _BLK

    users_r = users.astype(jnp.int32).reshape(_NW, b_per_w)
    movies_r = movies.astype(jnp.int32).reshape(_NW, b_per_w)

    utab_t = user_table.T
    mtab_t = movie_table.T
    wu = W.reshape(-1)[:_EMBED].astype(jnp.float32).reshape(_EMBED, 1)
    wm = W.reshape(-1)[_EMBED:].astype(jnp.float32).reshape(_EMBED, 1)

    scores_u = _tc_scores(utab_t, wu)
    scores_m = _tc_scores(mtab_t, wm)

    brep = jnp.full((_L,), b.reshape(()), dtype=jnp.float32)

    mesh = plsc.VectorSubcoreMesh(core_axis_name="c", subcore_axis_name="s")
    body = functools.partial(_sc_body, b_per_w=b_per_w, n_blocks=n_blocks)
    run = pl.kernel(
        body,
        out_type=jax.ShapeDtypeStruct((_NW, b_per_w), jnp.float32),
        mesh=mesh,
        compiler_params=pltpu.CompilerParams(needs_layout_passes=False),
        scratch_types=[
            pltpu.VMEM((b_per_w,), jnp.int32),
            pltpu.VMEM((b_per_w,), jnp.int32),
            pltpu.VMEM((b_per_w,), jnp.int32),
            pltpu.VMEM((b_per_w,), jnp.int32),
            pltpu.VMEM((2, _BLK, 128), jnp.float32),
            pltpu.VMEM((2, _BLK, 128), jnp.float32),
            pltpu.VMEM((b_per_w,), jnp.float32),
            pltpu.VMEM((_L,), jnp.float32),
            pltpu.SemaphoreType.DMA,
            pltpu.SemaphoreType.DMA,
            pltpu.SemaphoreType.DMA,
            pltpu.SemaphoreType.DMA,
        ],
    )
    out = run(users_r, movies_r, scores_u, scores_m, brep)
    return out.reshape(B, 1)


# speedup vs baseline: 2.0993x; 1.0141x over previous
import functools

import jax
import jax.numpy as jnp
from jax import lax
from jax.experimental import pallas as pl
from jax.experimental.pallas import tpu as pltpu
from jax.experimental.pallas import tpu_sc as plsc

_NC = 2
_NS = 16
_NW = _NC * _NS
_L = 16

_EMBED = 64
_BLK = 64
_C = 32768


def _tc_scores_body(xu_ref, xm_ref, wu_ref, wm_ref, ou_ref, om_ref):
    ou_ref[...] = jnp.sum(xu_ref[...] * wu_ref[...], axis=0).reshape(
        _C // 128, 128)
    om_ref[...] = jnp.sum(xm_ref[...] * wm_ref[...], axis=0).reshape(
        _C // 128, 128)


def _tc_scores(utab_t, mtab_t, wu, wm):
    n = utab_t.shape[1]
    grid = (n + _C - 1) // _C
    out = jax.ShapeDtypeStruct((grid * (_C // 128), 128), jnp.float32)
    return pl.pallas_call(
        _tc_scores_body,
        grid=(grid,),
        in_specs=[
            pl.BlockSpec((_EMBED, _C), lambda i: (0, i)),
            pl.BlockSpec((_EMBED, _C), lambda i: (0, i)),
            pl.BlockSpec((_EMBED, 1), lambda i: (0, 0)),
            pl.BlockSpec((_EMBED, 1), lambda i: (0, 0)),
        ],
        out_specs=[
            pl.BlockSpec((_C // 128, 128), lambda i: (i, 0)),
            pl.BlockSpec((_C // 128, 128), lambda i: (i, 0)),
        ],
        out_shape=[out, out],
    )(utab_t, mtab_t, wu, wm)


def _sc_body(users_r, movies_r, su, sm, brep, out,
             qu_v, tu_v, qm_v, tm_v, rows_u, rows_m, out_v, b_v,
             sem_u0, sem_u1, sem_m0, sem_m1,
             *, b_per_w, n_blocks):
    wid = lax.axis_index("s") * _NC + lax.axis_index("c")
    chunks_per_blk = _BLK // _L

    pltpu.sync_copy(users_r.at[wid], qu_v)
    pltpu.sync_copy(movies_r.at[wid], qm_v)
    pltpu.sync_copy(brep, b_v)

    def split(i, carry):
        sl = pl.ds(i * _L, _L)
        u = qu_v[sl]
        m = qm_v[sl]
        tu_v[sl] = u & 127
        qu_v[sl] = u >> 7
        tm_v[sl] = m & 127
        qm_v[sl] = m >> 7
        return carry

    lax.fori_loop(0, b_per_w // _L, split, 0)

    sems_u = (sem_u0, sem_u1)
    sems_m = (sem_m0, sem_m1)

    def fire(j):
        s = j % 2
        blk = pl.ds(j * _BLK, _BLK)
        pltpu.async_copy(su.at[qu_v.at[blk]], rows_u.at[s], sems_u[s])
        pltpu.async_copy(sm.at[qm_v.at[blk]], rows_m.at[s], sems_m[s])

    def drain(j):
        s = j % 2
        pltpu.make_async_copy(su.at[pl.ds(0, _BLK)], rows_u.at[s],
                              sems_u[s]).wait()
        pltpu.make_async_copy(sm.at[pl.ds(0, _BLK)], rows_m.at[s],
                              sems_m[s]).wait()

    row16 = lax.iota(jnp.int32, _L)
    bias = b_v[...]

    fire(0)
    fire(1)
    for j in range(n_blocks):
        s = j % 2
        drain(j)
        ru = rows_u.at[s]
        rm = rows_m.at[s]

        def chunk(k, carry, *, j=j, ru=ru, rm=rm):
            g = j * chunks_per_blk + k
            rid = k * _L + row16
            t_u = tu_v[pl.ds(g * _L, _L)]
            t_m = tm_v[pl.ds(g * _L, _L)]
            vu = plsc.load_gather(ru, [rid, t_u])
            vm = plsc.load_gather(rm, [rid, t_m])
            out_v[pl.ds(g * _L, _L)] = vu + vm + bias
            return carry

        lax.fori_loop(0, chunks_per_blk, chunk, 0)
        if j + 2 < n_blocks:
            fire(j + 2)

    pltpu.sync_copy(out_v, out.at[wid])


def kernel(users, movies, user_table, movie_table, W, b):
    B = users.shape[0]
    assert B % (_NW * _BLK) == 0
    b_per_w = B // _NW
    n_blocks = b_per_w // _BLK

    users_r = users.astype(jnp.int32).reshape(_NW, b_per_w)
    movies_r = movies.astype(jnp.int32).reshape(_NW, b_per_w)

    utab_t = user_table.T
    mtab_t = movie_table.T
    wu = W.reshape(-1)[:_EMBED].astype(jnp.float32).reshape(_EMBED, 1)
    wm = W.reshape(-1)[_EMBED:].astype(jnp.float32).reshape(_EMBED, 1)

    scores_u, scores_m = _tc_scores(utab_t, mtab_t, wu, wm)

    brep = jnp.full((_L,), b.reshape(()), dtype=jnp.float32)

    mesh = plsc.VectorSubcoreMesh(core_axis_name="c", subcore_axis_name="s")
    body = functools.partial(_sc_body, b_per_w=b_per_w, n_blocks=n_blocks)
    run = pl.kernel(
        body,
        out_type=jax.ShapeDtypeStruct((_NW, b_per_w), jnp.float32),
        mesh=mesh,
        compiler_params=pltpu.CompilerParams(needs_layout_passes=False),
        scratch_types=[
            pltpu.VMEM((b_per_w,), jnp.int32),
            pltpu.VMEM((b_per_w,), jnp.int32),
            pltpu.VMEM((b_per_w,), jnp.int32),
            pltpu.VMEM((b_per_w,), jnp.int32),
            pltpu.VMEM((2, _BLK, 128), jnp.float32),
            pltpu.VMEM((2, _BLK, 128), jnp.float32),
            pltpu.VMEM((b_per_w,), jnp.float32),
            pltpu.VMEM((_L,), jnp.float32),
            pltpu.SemaphoreType.DMA,
            pltpu.SemaphoreType.DMA,
            pltpu.SemaphoreType.DMA,
            pltpu.SemaphoreType.DMA,
        ],
    )
    out = run(users_r, movies_r, scores_u, scores_m, brep)
    return out.reshape(B, 1)


# speedup vs baseline: 2.1057x; 1.0030x over previous
import functools

import jax
import jax.numpy as jnp
from jax import lax
from jax.experimental import pallas as pl
from jax.experimental.pallas import tpu as pltpu
from jax.experimental.pallas import tpu_sc as plsc

_NC = 2
_NS = 16
_NW = _NC * _NS
_L = 16

_EMBED = 64
_BLK = 64
_C = 24576


def _tc_scores_body(xu_ref, xm_ref, wu_ref, wm_ref, ou_ref, om_ref):
    ou_ref[...] = jnp.sum(xu_ref[...] * wu_ref[...], axis=0).reshape(
        _C // 128, 128)
    om_ref[...] = jnp.sum(xm_ref[...] * wm_ref[...], axis=0).reshape(
        _C // 128, 128)


def _tc_scores(utab_t, mtab_t, wu, wm):
    n = utab_t.shape[1]
    grid = (n + _C - 1) // _C
    out = jax.ShapeDtypeStruct((grid * (_C // 128), 128), jnp.float32)
    return pl.pallas_call(
        _tc_scores_body,
        grid=(grid,),
        in_specs=[
            pl.BlockSpec((_EMBED, _C), lambda i: (0, i)),
            pl.BlockSpec((_EMBED, _C), lambda i: (0, i)),
            pl.BlockSpec((_EMBED, 1), lambda i: (0, 0)),
            pl.BlockSpec((_EMBED, 1), lambda i: (0, 0)),
        ],
        out_specs=[
            pl.BlockSpec((_C // 128, 128), lambda i: (i, 0)),
            pl.BlockSpec((_C // 128, 128), lambda i: (i, 0)),
        ],
        out_shape=[out, out],
    )(utab_t, mtab_t, wu, wm)


def _sc_body(users_r, movies_r, su, sm, brep, out,
             qu_v, tu_v, qm_v, tm_v, rows_u, rows_m, out_v, b_v,
             sem_u0, sem_u1, sem_m0, sem_m1,
             *, b_per_w, n_blocks):
    wid = lax.axis_index("s") * _NC + lax.axis_index("c")
    chunks_per_blk = _BLK // _L

    pltpu.sync_copy(users_r.at[wid], qu_v)
    pltpu.sync_copy(movies_r.at[wid], qm_v)
    pltpu.sync_copy(brep, b_v)

    def split(i, carry):
        sl = pl.ds(i * _L, _L)
        u = qu_v[sl]
        m = qm_v[sl]
        tu_v[sl] = u & 127
        qu_v[sl] = u >> 7
        tm_v[sl] = m & 127
        qm_v[sl] = m >> 7
        return carry

    lax.fori_loop(0, b_per_w // _L, split, 0)

    sems_u = (sem_u0, sem_u1)
    sems_m = (sem_m0, sem_m1)

    def fire(j):
        s = j % 2
        blk = pl.ds(j * _BLK, _BLK)
        pltpu.async_copy(su.at[qu_v.at[blk]], rows_u.at[s], sems_u[s])
        pltpu.async_copy(sm.at[qm_v.at[blk]], rows_m.at[s], sems_m[s])

    def drain(j):
        s = j % 2
        pltpu.make_async_copy(su.at[pl.ds(0, _BLK)], rows_u.at[s],
                              sems_u[s]).wait()
        pltpu.make_async_copy(sm.at[pl.ds(0, _BLK)], rows_m.at[s],
                              sems_m[s]).wait()

    row16 = lax.iota(jnp.int32, _L)
    bias = b_v[...]

    fire(0)
    fire(1)
    for j in range(n_blocks):
        s = j % 2
        drain(j)
        ru = rows_u.at[s]
        rm = rows_m.at[s]

        def chunk(k, carry, *, j=j, ru=ru, rm=rm):
            g = j * chunks_per_blk + k
            rid = k * _L + row16
            t_u = tu_v[pl.ds(g * _L, _L)]
            t_m = tm_v[pl.ds(g * _L, _L)]
            vu = plsc.load_gather(ru, [rid, t_u])
            vm = plsc.load_gather(rm, [rid, t_m])
            out_v[pl.ds(g * _L, _L)] = vu + vm + bias
            return carry

        lax.fori_loop(0, chunks_per_blk, chunk, 0)
        if j + 2 < n_blocks:
            fire(j + 2)

    pltpu.sync_copy(out_v, out.at[wid])


def kernel(users, movies, user_table, movie_table, W, b):
    B = users.shape[0]
    assert B % (_NW * _BLK) == 0
    b_per_w = B // _NW
    n_blocks = b_per_w // _BLK

    users_r = users.astype(jnp.int32).reshape(_NW, b_per_w)
    movies_r = movies.astype(jnp.int32).reshape(_NW, b_per_w)

    utab_t = user_table.T
    mtab_t = movie_table.T
    wu = W.reshape(-1)[:_EMBED].astype(jnp.float32).reshape(_EMBED, 1)
    wm = W.reshape(-1)[_EMBED:].astype(jnp.float32).reshape(_EMBED, 1)

    scores_u, scores_m = _tc_scores(utab_t, mtab_t, wu, wm)

    brep = jnp.full((_L,), b.reshape(()), dtype=jnp.float32)

    mesh = plsc.VectorSubcoreMesh(core_axis_name="c", subcore_axis_name="s")
    body = functools.partial(_sc_body, b_per_w=b_per_w, n_blocks=n_blocks)
    run = pl.kernel(
        body,
        out_type=jax.ShapeDtypeStruct((_NW, b_per_w), jnp.float32),
        mesh=mesh,
        compiler_params=pltpu.CompilerParams(needs_layout_passes=False),
        scratch_types=[
            pltpu.VMEM((b_per_w,), jnp.int32),
            pltpu.VMEM((b_per_w,), jnp.int32),
            pltpu.VMEM((b_per_w,), jnp.int32),
            pltpu.VMEM((b_per_w,), jnp.int32),
            pltpu.VMEM((2, _BLK, 128), jnp.float32),
            pltpu.VMEM((2, _BLK, 128), jnp.float32),
            pltpu.VMEM((b_per_w,), jnp.float32),
            pltpu.VMEM((_L,), jnp.float32),
            pltpu.SemaphoreType.DMA,
            pltpu.SemaphoreType.DMA,
            pltpu.SemaphoreType.DMA,
            pltpu.SemaphoreType.DMA,
        ],
    )
    out = run(users_r, movies_r, scores_u, scores_m, brep)
    return out.reshape(B, 1)
